# baseline (speedup 1.0000x reference)
import functools

import jax
import jax.numpy as jnp
from jax import lax
from jax.experimental import pallas as pl
from jax.experimental.pallas import tpu as pltpu
from jax.experimental.pallas import tpu_sc as plsc

_LANES = 16


def _sc_call(n_total, n_rows, n_bound, width, chunk):
  info = plsc.get_sparse_core_info()
  num_cores, num_subcores = info.num_cores, info.num_subcores
  nw = num_cores * num_subcores
  per_w = n_total // nw
  n_chunks = per_w // chunk
  n_vec = chunk // _LANES
  n_gather = chunk // 128
  scale = float(n_bound - 1)
  nbins_pad = ((n_bound + _LANES) // _LANES) * _LANES

  mesh = plsc.VectorSubcoreMesh(
      core_axis_name="c", subcore_axis_name="s", num_cores=num_cores,
      num_subcores=num_subcores)

  @functools.partial(
      pl.kernel,
      out_type=jax.ShapeDtypeStruct((n_total, width + 1), jnp.float32),
      mesh=mesh,
      scratch_types=[
          pltpu.VMEM((nbins_pad,), jnp.float32),
          pltpu.VMEM((chunk,), jnp.float32),
          pltpu.VMEM((chunk,), jnp.float32),
          pltpu.VMEM((chunk, 1), jnp.float32),
          pltpu.VMEM((chunk,), jnp.int32),
          pltpu.VMEM((chunk,), jnp.int32),
          pltpu.VMEM((chunk, width), jnp.float32),
          pltpu.VMEM(((n_rows + 1) * width,), jnp.float32),
          pltpu.SemaphoreType.DMA,
      ],
      compiler_params=pltpu.CompilerParams(
          needs_layout_passes=False, use_tc_tiling_on_sc=False),
  )
  def call(pitch_hbm, vflag_hbm, mask_hbm, bins_hbm, table_hbm, out_hbm,
           bins_v, pitch_v, vflag_v, flagcol_v, mask_v, idx_v, rows_v,
           table_v, sem):
    wid = lax.axis_index("s") * num_cores + lax.axis_index("c")
    pltpu.sync_copy(bins_hbm, bins_v.at[pl.ds(0, n_bound)])
    pltpu.sync_copy(table_hbm, table_v)
    lane = lax.broadcasted_iota(jnp.int32, (_LANES,), 0)
    zero16 = jnp.zeros((_LANES,), jnp.int32)

    for c in range(n_chunks):
      base = wid * per_w + c * chunk
      pltpu.sync_copy(pitch_hbm.at[pl.ds(base, chunk)], pitch_v)
      pltpu.sync_copy(vflag_hbm.at[pl.ds(base, chunk)], vflag_v)
      pltpu.sync_copy(mask_hbm.at[pl.ds(base, chunk)], mask_v)

      def bucketize(i, _):
        sl = pl.ds(i * _LANES, _LANES)
        p = pitch_v[sl]
        guess = (p * scale).astype(jnp.int32)
        lo = jnp.clip(guess - 1, 0, n_bound - 3)
        b0 = plsc.load_gather(bins_v, [lo])
        b1 = plsc.load_gather(bins_v, [lo + 1])
        b2 = plsc.load_gather(bins_v, [lo + 2])
        cnt = ((b0 < p).astype(jnp.int32) + (b1 < p).astype(jnp.int32)
               + (b2 < p).astype(jnp.int32))
        m = mask_v[sl]
        idx_v[sl] = jnp.where(m != 0, n_rows, lo + cnt)
        vf = jnp.where(m != 0, 0.0, vflag_v[sl])
        plsc.store_scatter(flagcol_v, [i * _LANES + lane, zero16], vf)
        return 0

      lax.fori_loop(0, n_vec, bucketize, 0)

      def gather_rows(i, _):
        r16 = i * _LANES + lane
        ibase = idx_v[pl.ds(i * _LANES, _LANES)] * width
        for f in range(width):
          vals = plsc.load_gather(table_v, [ibase + f])
          plsc.store_scatter(rows_v, [r16, jnp.full((_LANES,), f, jnp.int32)],
                             vals)
        return 0

      lax.fori_loop(0, n_vec, gather_rows, 0)

      pltpu.sync_copy(rows_v, out_hbm.at[pl.ds(base, chunk), pl.ds(0, width)])
      pltpu.sync_copy(flagcol_v,
                      out_hbm.at[pl.ds(base, chunk), pl.ds(width, 1)])

  return call


def kernel(pitch, v_flag, pitch_mask, pitch_bins, emb_table):
  b, t = pitch.shape
  n_rows, width = emb_table.shape
  n_bound = pitch_bins.shape[0]
  n_total = b * t
  call = _sc_call(n_total, n_rows, n_bound, width, chunk=512)
  table_p = jnp.pad(emb_table, ((0, 1), (0, 0))).reshape(-1)
  out = call(
      pitch.reshape(-1),
      v_flag.reshape(-1),
      pitch_mask.reshape(-1).astype(jnp.int32),
      pitch_bins,
      table_p,
  )
  return out.reshape(b, t, width + 1)


# speedup vs baseline: 1.7761x; 1.7761x over previous
import functools

import jax
import jax.numpy as jnp
from jax import lax
from jax.experimental import pallas as pl
from jax.experimental.pallas import tpu as pltpu
from jax.experimental.pallas import tpu_sc as plsc

_LANES = 16


def _sc_call(n_total, n_rows, n_bound, width, chunk):
  info = plsc.get_sparse_core_info()
  num_cores, num_subcores = info.num_cores, info.num_subcores
  nw = num_cores * num_subcores
  per_w = n_total // nw
  n_chunks = per_w // chunk
  n_vec = chunk // _LANES
  scale = float(n_bound - 1)
  nbins_pad = ((n_bound + _LANES) // _LANES) * _LANES

  mesh = plsc.VectorSubcoreMesh(
      core_axis_name="c", subcore_axis_name="s", num_cores=num_cores,
      num_subcores=num_subcores)

  @functools.partial(
      pl.kernel,
      out_type=jax.ShapeDtypeStruct((width + 1, n_total), jnp.float32),
      mesh=mesh,
      scratch_types=[
          pltpu.VMEM((nbins_pad,), jnp.float32),
          pltpu.VMEM((per_w,), jnp.float32),
          pltpu.VMEM((per_w,), jnp.float32),
          pltpu.VMEM((per_w,), jnp.int32),
          pltpu.VMEM((per_w,), jnp.int32),
          pltpu.VMEM((width + 1, chunk), jnp.float32),
          pltpu.VMEM((width + 1, chunk), jnp.float32),
          pltpu.VMEM(((n_rows + 1) * width,), jnp.float32),
          pltpu.SemaphoreType.DMA,
          pltpu.SemaphoreType.DMA,
          pltpu.SemaphoreType.DMA,
      ],
      compiler_params=pltpu.CompilerParams(
          needs_layout_passes=False, use_tc_tiling_on_sc=False),
  )
  def call(pitch_hbm, vflag_hbm, mask_hbm, bins_hbm, table_hbm, out_hbm,
           bins_v, pitch_v, vflag_v, mask_v, idx_v, cols0_v, cols1_v,
           table_v, sem_t, sem_in, sem_out):
    wid = lax.axis_index("s") * num_cores + lax.axis_index("c")
    span = wid * per_w
    lane = lax.broadcasted_iota(jnp.int32, (_LANES,), 0)
    del lane

    h_table = pltpu.async_copy(table_hbm, table_v, sem_t)
    h_in = [
        pltpu.async_copy(pitch_hbm.at[pl.ds(span, per_w)], pitch_v, sem_in),
        pltpu.async_copy(vflag_hbm.at[pl.ds(span, per_w)], vflag_v, sem_in),
        pltpu.async_copy(mask_hbm.at[pl.ds(span, per_w)], mask_v, sem_in),
    ]
    pltpu.sync_copy(bins_hbm, bins_v.at[pl.ds(0, n_bound)])
    for h in h_in:
      h.wait()

    def bucketize(i, _):
      sl = pl.ds(i * _LANES, _LANES)
      p = pitch_v[sl]
      guess = (p * scale).astype(jnp.int32)
      lo = jnp.clip(guess - 1, 0, n_bound - 3)
      b0 = plsc.load_gather(bins_v, [lo])
      b1 = plsc.load_gather(bins_v, [lo + 1])
      b2 = plsc.load_gather(bins_v, [lo + 2])
      cnt = ((b0 < p).astype(jnp.int32) + (b1 < p).astype(jnp.int32)
             + (b2 < p).astype(jnp.int32))
      m = mask_v[sl]
      idx_v[sl] = jnp.where(m != 0, n_rows, lo + cnt) * width
      vflag_v[sl] = jnp.where(m != 0, 0.0, vflag_v[sl])
      return 0

    lax.fori_loop(0, per_w // _LANES, bucketize, 0)
    h_table.wait()

    bufs = (cols0_v, cols1_v)
    handles = []
    for c in range(n_chunks):
      cols_v = bufs[c % 2]
      if c >= 2:
        handles[c - 2].wait()

      def gather_cols(t, _):
        i = t >> 4
        fo = (t & 15) * 8
        sl16 = pl.ds(i * _LANES, _LANES)
        ibase = idx_v[pl.ds(c * chunk + i * _LANES, _LANES)] + fo
        for k in range(8):
          cols_v[fo + k, sl16] = plsc.load_gather(table_v, [ibase + k])
        return 0

      lax.fori_loop(0, n_vec * 16, gather_cols, 0)

      def flag_row(i, _):
        sl16 = pl.ds(i * _LANES, _LANES)
        cols_v[width, sl16] = vflag_v[pl.ds(c * chunk + i * _LANES, _LANES)]
        return 0

      lax.fori_loop(0, n_vec, flag_row, 0)
      handles.append(
          pltpu.async_copy(cols_v,
                           out_hbm.at[:, pl.ds(span + c * chunk, chunk)],
                           sem_out))
    for h in handles[-2:]:
      h.wait()

  return call


def kernel(pitch, v_flag, pitch_mask, pitch_bins, emb_table):
  b, t = pitch.shape
  n_rows, width = emb_table.shape
  n_bound = pitch_bins.shape[0]
  n_total = b * t
  call = _sc_call(n_total, n_rows, n_bound, width, chunk=256)
  table_p = jnp.pad(emb_table, ((0, 1), (0, 0))).reshape(-1)
  out = call(
      pitch.reshape(-1),
      v_flag.reshape(-1),
      pitch_mask.reshape(-1).astype(jnp.int32),
      pitch_bins,
      table_p,
  )
  return jnp.transpose(out.reshape(width + 1, b, t), (1, 2, 0))


# speedup vs baseline: 10.2156x; 5.7517x over previous
import functools

import jax
import jax.numpy as jnp
from jax import lax
from jax.experimental import pallas as pl
from jax.experimental.pallas import tpu as pltpu
from jax.experimental.pallas import tpu_sc as plsc

_LANES = 16


def _sc_call(n_total, n_rows, n_bound, width, chunk):
  info = plsc.get_sparse_core_info()
  num_cores, num_subcores = info.num_cores, info.num_subcores
  nw = num_cores * num_subcores
  per_w = n_total // nw
  n_chunks = per_w // chunk
  n_vec = chunk // _LANES
  scale = float(n_bound - 1)
  nbins_pad = ((n_bound + _LANES) // _LANES) * _LANES

  mesh = plsc.VectorSubcoreMesh(
      core_axis_name="c", subcore_axis_name="s", num_cores=num_cores,
      num_subcores=num_subcores)

  @functools.partial(
      pl.kernel,
      out_type=jax.ShapeDtypeStruct((width + 1, n_total), jnp.float32),
      mesh=mesh,
      scratch_types=[
          pltpu.VMEM((nbins_pad,), jnp.float32),
          pltpu.VMEM((per_w,), jnp.float32),
          pltpu.VMEM((per_w,), jnp.float32),
          pltpu.VMEM((per_w,), jnp.int32),
          pltpu.VMEM((per_w,), jnp.int32),
          pltpu.VMEM((width + 1, chunk), jnp.float32),
          pltpu.VMEM((width + 1, chunk), jnp.float32),
          pltpu.VMEM(((n_rows + 1) * (width + 1) + 7,), jnp.float32),
          pltpu.SemaphoreType.DMA,
          pltpu.SemaphoreType.DMA,
          pltpu.SemaphoreType.DMA,
      ],
      compiler_params=pltpu.CompilerParams(
          needs_layout_passes=False, use_tc_tiling_on_sc=False),
  )
  def call(pitch_hbm, vflag_hbm, mask_hbm, bins_hbm, table_hbm, out_hbm,
           bins_v, pitch_v, vflag_v, mask_v, idx_v, cols0_v, cols1_v,
           table_v, sem_t, sem_in, sem_out):
    wid = lax.axis_index("s") * num_cores + lax.axis_index("c")
    span = wid * per_w
    lane = lax.broadcasted_iota(jnp.int32, (_LANES,), 0)
    del lane

    h_table = pltpu.async_copy(table_hbm, table_v, sem_t)
    h_in = [
        pltpu.async_copy(pitch_hbm.at[pl.ds(span, per_w)], pitch_v, sem_in),
        pltpu.async_copy(vflag_hbm.at[pl.ds(span, per_w)], vflag_v, sem_in),
        pltpu.async_copy(mask_hbm.at[pl.ds(span, per_w)], mask_v, sem_in),
    ]
    pltpu.sync_copy(bins_hbm, bins_v.at[pl.ds(0, n_bound)])
    for h in h_in:
      h.wait()

    @plsc.parallel_loop(0, per_w // _LANES, unroll=2)
    def bucketize(i):
      sl = pl.ds(i * _LANES, _LANES)
      p = pitch_v[sl]
      guess = (p * scale).astype(jnp.int32)
      lo = jnp.clip(guess - 1, 0, n_bound - 3)
      b0 = plsc.load_gather(bins_v, [lo])
      b1 = plsc.load_gather(bins_v, [lo + 1])
      b2 = plsc.load_gather(bins_v, [lo + 2])
      cnt = ((b0 < p).astype(jnp.int32) + (b1 < p).astype(jnp.int32)
             + (b2 < p).astype(jnp.int32))
      m = mask_v[sl]
      idx_v[sl] = (jnp.where(m != 0, n_rows, lo + cnt)
                   * (width + 1))
      vflag_v[sl] = jnp.where(m != 0, 0.0, vflag_v[sl])
    h_table.wait()

    bufs = (cols0_v, cols1_v)
    handles = []
    for c in range(n_chunks):
      cols_v = bufs[c % 2]
      if c >= 2:
        handles[c - 2].wait()

      @plsc.parallel_loop(0, n_vec * 16, unroll=2)
      def gather_cols(t):
        i = t >> 4
        fo = (t & 15) * 8
        sl16 = pl.ds(i * _LANES, _LANES)
        ibase = idx_v[pl.ds(c * chunk + i * _LANES, _LANES)] + fo
        for k in range(8):
          cols_v[fo + k, sl16] = plsc.load_gather(table_v, [ibase + k])

      @plsc.parallel_loop(0, n_vec)
      def flag_row(i):
        sl16 = pl.ds(i * _LANES, _LANES)
        cols_v[width, sl16] = vflag_v[pl.ds(c * chunk + i * _LANES, _LANES)]
      handles.append(
          pltpu.async_copy(cols_v,
                           out_hbm.at[:, pl.ds(span + c * chunk, chunk)],
                           sem_out))
    for h in handles[-2:]:
      h.wait()

  return call


def kernel(pitch, v_flag, pitch_mask, pitch_bins, emb_table):
  b, t = pitch.shape
  n_rows, width = emb_table.shape
  n_bound = pitch_bins.shape[0]
  n_total = b * t
  call = _sc_call(n_total, n_rows, n_bound, width, chunk=256)
  table_p = jnp.pad(emb_table, ((0, 1), (0, 1))).reshape(-1)
  table_p = jnp.pad(table_p, (0, 7))

  gb, br, gt, tc = b // 8, 8, t // 128, 128

  def tiled_flat(x):
    return x.reshape(gb, br, gt, tc).transpose(0, 2, 1, 3).reshape(-1)

  out = call(
      tiled_flat(pitch),
      tiled_flat(v_flag),
      tiled_flat(pitch_mask.reshape(b, t).astype(jnp.int32)),
      pitch_bins,
      table_p,
  )
  out = out.reshape(width + 1, gb, gt, br, tc)
  return jnp.transpose(out, (1, 3, 2, 4, 0)).reshape(b, t, width + 1)
